# baseline (speedup 1.0000x reference)
import functools
import math

import jax
import jax.numpy as jnp
from jax.experimental import pallas as pl
from jax.experimental.pallas import tpu as pltpu

N = 10000
NFEAT = 128
NHID = 64
NCLASS = 32
EPS = 1e-5

MB = 1000
KB = 512
NM = N // MB
NK = -(-N // KB)
NPAD = NK * KB


def _gcn_body(adj_ref, adj1_ref, adj2_ref, x_ref, w1_ref, b1_ref, b2_ref,
              fw1_ref, fb1_ref, fw2_ref, fb2_ref, bn1g_ref, bn1b_ref,
              bn2g_ref, bn2b_ref, w2_ref, mw_ref, mb_ref,
              o_ref, acc_ref, z1_ref, z2_ref):
    l = pl.program_id(0)
    i = pl.program_id(1)
    k = pl.program_id(2)

    @pl.when(k == 0)
    def _init():
        acc_ref[...] = jnp.zeros_like(acc_ref)

    @pl.when((l == 0) & (i == 0))
    def _fill_z1():
        s = jnp.minimum(k * KB, N - KB)
        z1_ref[pl.ds(s, KB), :] = jnp.dot(
            x_ref[pl.ds(s, KB), :], w1_ref[...],
            preferred_element_type=jnp.float32)

    zsrc1 = z1_ref[pl.ds(k * KB, KB), :]
    zsrc2 = z2_ref[pl.ds(k * KB, KB), :]
    z = jnp.where(l == 0, zsrc1, zsrc2)
    valid = jax.lax.broadcasted_iota(jnp.int32, z.shape, 0) < (N - k * KB)
    z = jnp.where(valid, z, 0.0)
    p0 = jnp.dot(adj_ref[...], z[:, 0:NHID],
                 preferred_element_type=jnp.float32)
    p1 = jnp.dot(adj1_ref[...], z[:, NHID:2 * NHID],
                 preferred_element_type=jnp.float32)
    p2 = jnp.dot(adj2_ref[...], z[:, 2 * NHID:3 * NHID],
                 preferred_element_type=jnp.float32)
    acc_ref[...] += jnp.concatenate([p0, p1, p2], axis=1)

    @pl.when(k == NK - 1)
    def _epilogue():
        b = jnp.where(l == 0, b1_ref[...], b2_ref[...])
        fw = jnp.where(l == 0, fw1_ref[...], fw2_ref[...])
        fb = jnp.where(l == 0, fb1_ref[...], fb2_ref[...])
        bng = jnp.where(l == 0, bn1g_ref[...], bn2g_ref[...])
        bnb = jnp.where(l == 0, bn1b_ref[...], bn2b_ref[...])
        xcat = jnp.maximum(acc_ref[...] + b, 0.0)
        h = jnp.dot(xcat, fw, preferred_element_type=jnp.float32) + fb
        h = h * (bng * (1.0 / math.sqrt(1.0 + EPS))) + bnb

        @pl.when(l == 0)
        def _mid():
            z2_ref[pl.ds(i * MB, MB), :] = jnp.dot(
                h, w2_ref[...], preferred_element_type=jnp.float32)

        @pl.when(l == 1)
        def _head():
            o = jnp.dot(h, mw_ref[...],
                        preferred_element_type=jnp.float32) + mb_ref[...]
            m = jnp.max(o, axis=1, keepdims=True)
            s = o - m
            lse = jnp.log(jnp.sum(jnp.exp(s), axis=1, keepdims=True))
            o_ref[...] = s - lse


def kernel(x, adj, adj1, adj2, gc11_w, gc11_b, gc12_w, gc12_b, gc13_w,
           gc13_b, gc21_w, gc21_b, gc22_w, gc22_b, gc23_w, gc23_b,
           fusion_w, fusion_b, fusion1_w, fusion1_b, mlp1_w, mlp1_b,
           bn1_g, bn1_b, bn2_g, bn2_b):
    f32 = jnp.float32
    w1cat = jnp.concatenate([gc11_w, gc12_w, gc13_w], axis=1)
    b1cat = jnp.concatenate([gc11_b, gc12_b, gc13_b])[None, :]
    w2cat = jnp.concatenate([gc21_w, gc22_w, gc23_w], axis=1)
    b2cat = jnp.concatenate([gc21_b, gc22_b, gc23_b])[None, :]

    adjspec = pl.BlockSpec((MB, KB), lambda l, i, k: (i, k))
    small = lambda r, c: pl.BlockSpec((r, c), lambda l, i, k: (0, 0))

    out = pl.pallas_call(
        _gcn_body,
        grid=(2, NM, NK),
        in_specs=[
            adjspec, adjspec, adjspec,
            small(N, NFEAT),
            small(NFEAT, 3 * NHID),
            small(1, 3 * NHID),
            small(1, 3 * NHID),
            small(3 * NHID, NHID),
            small(1, NHID),
            small(3 * NHID, NHID),
            small(1, NHID),
            small(1, NHID),
            small(1, NHID),
            small(1, NHID),
            small(1, NHID),
            small(NHID, 3 * NHID),
            small(NHID, NCLASS),
            small(1, NCLASS),
        ],
        out_specs=pl.BlockSpec((MB, NCLASS), lambda l, i, k: (l * i, 0)),
        out_shape=jax.ShapeDtypeStruct((N, NCLASS), f32),
        scratch_shapes=[pltpu.VMEM((MB, 3 * NHID), f32),
                        pltpu.VMEM((NPAD, 3 * NHID), f32),
                        pltpu.VMEM((NPAD, 3 * NHID), f32)],
        compiler_params=pltpu.CompilerParams(
            dimension_semantics=("arbitrary", "arbitrary", "arbitrary"),
            vmem_limit_bytes=100 * 1024 * 1024),
    )(adj, adj1, adj2, x, w1cat, b1cat, b2cat, fusion_w, fusion_b[None, :],
      fusion1_w, fusion1_b[None, :], bn1_g[None, :], bn1_b[None, :],
      bn2_g[None, :], bn2_b[None, :], w2cat, mlp1_w, mlp1_b[None, :])

    return out


# speedup vs baseline: 1.1649x; 1.1649x over previous
import functools
import math

import jax
import jax.numpy as jnp
from jax.experimental import pallas as pl
from jax.experimental.pallas import tpu as pltpu

N = 10000
NFEAT = 128
NHID = 64
NCLASS = 32
EPS = 1e-5

MB = 1000
KB = 1024
NM = N // MB
NK = -(-N // KB)
NPAD = NK * KB


def _gcn_body(adj_ref, adj1_ref, adj2_ref, x_ref, w1_ref, b1_ref, b2_ref,
              fw1_ref, fb1_ref, fw2_ref, fb2_ref, bn1g_ref, bn1b_ref,
              bn2g_ref, bn2b_ref, w2_ref, mw_ref, mb_ref,
              o_ref, acc_ref, z1_ref, z2_ref):
    l = pl.program_id(0)
    i = pl.program_id(1)
    k = pl.program_id(2)

    @pl.when(k == 0)
    def _init():
        acc_ref[...] = jnp.zeros_like(acc_ref)

    @pl.when((l == 0) & (i == 0))
    def _fill_z1():
        s = jnp.minimum(k * KB, N - KB)
        z1_ref[pl.ds(s, KB), :] = jnp.dot(
            x_ref[pl.ds(s, KB), :], w1_ref[...],
            preferred_element_type=jnp.float32)

    zsrc1 = z1_ref[pl.ds(k * KB, KB), :]
    zsrc2 = z2_ref[pl.ds(k * KB, KB), :]
    z = jnp.where(l == 0, zsrc1, zsrc2)
    valid = jax.lax.broadcasted_iota(jnp.int32, z.shape, 0) < (N - k * KB)
    z = jnp.where(valid, z, 0.0)
    p0 = jnp.dot(adj_ref[...], z[:, 0:NHID],
                 preferred_element_type=jnp.float32)
    p1 = jnp.dot(adj1_ref[...], z[:, NHID:2 * NHID],
                 preferred_element_type=jnp.float32)
    p2 = jnp.dot(adj2_ref[...], z[:, 2 * NHID:3 * NHID],
                 preferred_element_type=jnp.float32)
    acc_ref[...] += jnp.concatenate([p0, p1, p2], axis=1)

    @pl.when(k == NK - 1)
    def _epilogue():
        b = jnp.where(l == 0, b1_ref[...], b2_ref[...])
        fw = jnp.where(l == 0, fw1_ref[...], fw2_ref[...])
        fb = jnp.where(l == 0, fb1_ref[...], fb2_ref[...])
        bng = jnp.where(l == 0, bn1g_ref[...], bn2g_ref[...])
        bnb = jnp.where(l == 0, bn1b_ref[...], bn2b_ref[...])
        xcat = jnp.maximum(acc_ref[...] + b, 0.0)
        h = jnp.dot(xcat, fw, preferred_element_type=jnp.float32) + fb
        h = h * (bng * (1.0 / math.sqrt(1.0 + EPS))) + bnb

        @pl.when(l == 0)
        def _mid():
            z2_ref[pl.ds(i * MB, MB), :] = jnp.dot(
                h, w2_ref[...], preferred_element_type=jnp.float32)

        @pl.when(l == 1)
        def _head():
            o = jnp.dot(h, mw_ref[...],
                        preferred_element_type=jnp.float32) + mb_ref[...]
            m = jnp.max(o, axis=1, keepdims=True)
            s = o - m
            lse = jnp.log(jnp.sum(jnp.exp(s), axis=1, keepdims=True))
            o_ref[...] = s - lse


def kernel(x, adj, adj1, adj2, gc11_w, gc11_b, gc12_w, gc12_b, gc13_w,
           gc13_b, gc21_w, gc21_b, gc22_w, gc22_b, gc23_w, gc23_b,
           fusion_w, fusion_b, fusion1_w, fusion1_b, mlp1_w, mlp1_b,
           bn1_g, bn1_b, bn2_g, bn2_b):
    f32 = jnp.float32
    w1cat = jnp.concatenate([gc11_w, gc12_w, gc13_w], axis=1)
    b1cat = jnp.concatenate([gc11_b, gc12_b, gc13_b])[None, :]
    w2cat = jnp.concatenate([gc21_w, gc22_w, gc23_w], axis=1)
    b2cat = jnp.concatenate([gc21_b, gc22_b, gc23_b])[None, :]

    adjspec = pl.BlockSpec((MB, KB), lambda l, i, k: (i, k))
    small = lambda r, c: pl.BlockSpec((r, c), lambda l, i, k: (0, 0))

    out = pl.pallas_call(
        _gcn_body,
        grid=(2, NM, NK),
        in_specs=[
            adjspec, adjspec, adjspec,
            small(N, NFEAT),
            small(NFEAT, 3 * NHID),
            small(1, 3 * NHID),
            small(1, 3 * NHID),
            small(3 * NHID, NHID),
            small(1, NHID),
            small(3 * NHID, NHID),
            small(1, NHID),
            small(1, NHID),
            small(1, NHID),
            small(1, NHID),
            small(1, NHID),
            small(NHID, 3 * NHID),
            small(NHID, NCLASS),
            small(1, NCLASS),
        ],
        out_specs=pl.BlockSpec((MB, NCLASS), lambda l, i, k: (l * i, 0)),
        out_shape=jax.ShapeDtypeStruct((N, NCLASS), f32),
        scratch_shapes=[pltpu.VMEM((MB, 3 * NHID), f32),
                        pltpu.VMEM((NPAD, 3 * NHID), f32),
                        pltpu.VMEM((NPAD, 3 * NHID), f32)],
        compiler_params=pltpu.CompilerParams(
            dimension_semantics=("arbitrary", "arbitrary", "arbitrary"),
            vmem_limit_bytes=100 * 1024 * 1024),
    )(adj, adj1, adj2, x, w1cat, b1cat, b2cat, fusion_w, fusion_b[None, :],
      fusion1_w, fusion1_b[None, :], bn1_g[None, :], bn1_b[None, :],
      bn2_g[None, :], bn2_b[None, :], w2cat, mlp1_w, mlp1_b[None, :])

    return out
